# baseline (speedup 1.0000x reference)
import functools

import jax
import jax.numpy as jnp
from jax import lax
from jax.experimental import pallas as pl
from jax.experimental.pallas import tpu as pltpu, tpu_sc as plsc

_ROWS = 16384
_COLS = 50
_CLASSES = 256
_TOTAL = _ROWS * _COLS
_N = _TOTAL * _CLASSES
_R2 = 6400
_C2 = 128
_BR = 32

_NC = 2
_NS = 16
_L = 16
_NW = _NC * _NS
_PER_W = _TOTAL // _NW
_IB = 128
_NI = _PER_W // _IB
_W = 32


def _zero_body(o_ref):
    o_ref[...] = jnp.zeros((_BR, _C2, _CLASSES), jnp.float32)


def _scatter_body(x_hbm, out_hbm, idx_v, offs_v, ones_v, sem):
    wid = lax.axis_index("s") * _NC + lax.axis_index("c")
    base = wid * _PER_W
    pltpu.sync_copy(x_hbm.at[pl.ds(base, _PER_W)], idx_v)

    lane = lax.iota(jnp.int32, _L)
    for c in range(_IB // _L):
        ones_v[pl.ds(c * _L, _L)] = jnp.ones((_L,), jnp.float32)

    def _fill(j, _):
        for c in range(_IB // _L):
            s = j * _IB + c * _L
            cols = idx_v[pl.ds(s, _L)]
            offs_v[j, pl.ds(c * _L, _L)] = (base + s + lane) * _CLASSES + cols
        return 0

    lax.fori_loop(0, _NI, _fill, 0)

    def _wait_one():
        pltpu.make_async_copy(ones_v, out_hbm.at[offs_v.at[0]], sem).wait()

    def _fire(j, _):
        pltpu.async_copy(ones_v, out_hbm.at[offs_v.at[j]], sem)

        @pl.when(j >= _W)
        def _():
            _wait_one()

        return 0

    lax.fori_loop(0, _NI, _fire, 0)

    def _drain(j, _):
        _wait_one()
        return 0

    lax.fori_loop(0, _W, _drain, 0)


@functools.cache
def _make_sc_scatter():
    return pl.kernel(
        _scatter_body,
        out_type=(),
        mesh=plsc.VectorSubcoreMesh(
            core_axis_name="c", subcore_axis_name="s",
            num_cores=_NC, num_subcores=_NS,
        ),
        scratch_types=[
            pltpu.VMEM((_PER_W,), jnp.int32),
            pltpu.VMEM((_NI, _IB), jnp.int32),
            pltpu.VMEM((_IB,), jnp.float32),
            pltpu.SemaphoreType.DMA,
        ],
    )


@jax.jit
def _onehot(x):
    zeros3 = pl.pallas_call(
        _zero_body,
        grid=(_R2 // _BR,),
        out_specs=pl.BlockSpec((_BR, _C2, _CLASSES), lambda i: (i, 0, 0)),
        out_shape=jax.ShapeDtypeStruct((_R2, _C2, _CLASSES), jnp.float32),
    )()
    r = jax.new_ref(zeros3.reshape(_N))
    _make_sc_scatter()(x.reshape(_TOTAL).astype(jnp.int32), r)
    return jax.freeze(r)


def kernel(x):
    return _onehot(x).reshape(_ROWS, _COLS, _CLASSES)
